# baseline (speedup 1.0000x reference)
import functools

import jax
import jax.numpy as jnp
from jax import lax
from jax.experimental import pallas as pl
from jax.experimental.pallas import tpu as pltpu
from jax.experimental.pallas import tpu_sc as plsc


def _gather3(B, D, NC, NS, R):
    NW = NC * NS
    b_per_w = B // NW
    mesh = plsc.VectorSubcoreMesh(core_axis_name="c", subcore_axis_name="s")

    @functools.partial(
        pl.kernel,
        mesh=mesh,
        out_type=(
            jax.ShapeDtypeStruct((B, D), jnp.float32),
            jax.ShapeDtypeStruct((B, D), jnp.float32),
            jax.ShapeDtypeStruct((B, D), jnp.float32),
        ),
        scratch_types=[
            pltpu.VMEM((3 * b_per_w,), jnp.int32),
            pltpu.VMEM((b_per_w, D), jnp.float32),
            pltpu.VMEM_SHARED((R, D), jnp.float32),
            pltpu.SemaphoreType.DMA,
            pltpu.SemaphoreType.DMA,
            pltpu.SemaphoreType.DMA,
        ],
    )
    def k(emb_hbm, kgg_hbm, rel_hbm, scg_ids_hbm, kgg_ids_hbm, rel_ids_hbm,
          out_scg, out_kgg, out_rel, idx_v, rows_v, rel_sh, isem, rsem, gsem):
        sid = lax.axis_index("s")
        wid = sid * NC + lax.axis_index("c")
        base = wid * b_per_w

        @pl.when(sid == 0)
        def _():
            pltpu.async_copy(rel_hbm, rel_sh, rsem)

        id_copies = [
            pltpu.async_copy(ids_hbm.at[pl.ds(base, b_per_w)],
                             idx_v.at[pl.ds(t * b_per_w, b_per_w)], isem)
            for t, ids_hbm in enumerate(
                (scg_ids_hbm, kgg_ids_hbm, rel_ids_hbm))
        ]
        for cp in id_copies:
            cp.wait()

        def do_table(table_ref, out_hbm, t):
            pltpu.async_copy(
                table_ref.at[idx_v.at[pl.ds(t * b_per_w, b_per_w)]],
                rows_v, gsem).wait()
            plsc.subcore_barrier()
            pltpu.sync_copy(rows_v, out_hbm.at[pl.ds(base, b_per_w)])
            plsc.subcore_barrier()

        do_table(emb_hbm, out_scg, 0)
        do_table(kgg_hbm, out_kgg, 1)

        @pl.when(sid == 0)
        def _():
            pltpu.make_async_copy(rel_hbm, rel_sh, rsem).wait()
        plsc.subcore_barrier()
        do_table(rel_sh, out_rel, 2)

    return k


def kernel(embedding, kgg_table, rel_table, scg_ids, relation_ids, kgg_ids):
    B = scg_ids.shape[0]
    D = embedding.shape[1]
    R = rel_table.shape[0]
    info = plsc.get_sparse_core_info()
    NC, NS = info.num_cores, info.num_subcores
    k = _gather3(B, D, NC, NS, R)
    if scg_ids.dtype != jnp.int32:
        scg_ids = scg_ids.astype(jnp.int32)
        relation_ids = relation_ids.astype(jnp.int32)
        kgg_ids = kgg_ids.astype(jnp.int32)
    scg, kgg, rel = k(embedding, kgg_table, rel_table,
                      scg_ids, kgg_ids, relation_ids)
    return (scg, kgg, rel)


# speedup vs baseline: 1.0419x; 1.0419x over previous
import functools

import jax
import jax.numpy as jnp
from jax import lax
from jax.experimental import pallas as pl
from jax.experimental.pallas import tpu as pltpu
from jax.experimental.pallas import tpu_sc as plsc


def _gather3(B, D, NC, NS, R):
    NW = NC * NS
    b_per_w = B // NW
    mesh = plsc.VectorSubcoreMesh(core_axis_name="c", subcore_axis_name="s")

    @functools.partial(
        pl.kernel,
        mesh=mesh,
        out_type=(
            jax.ShapeDtypeStruct((B, D), jnp.float32),
            jax.ShapeDtypeStruct((B, D), jnp.float32),
            jax.ShapeDtypeStruct((B, D), jnp.float32),
        ),
        scratch_types=[
            pltpu.VMEM((3 * b_per_w,), jnp.int32),
            pltpu.VMEM((b_per_w, D), jnp.float32),
            pltpu.VMEM((b_per_w // 2, D), jnp.float32),
            pltpu.VMEM_SHARED((R, D), jnp.float32),
            pltpu.SemaphoreType.DMA,
            pltpu.SemaphoreType.DMA,
            pltpu.SemaphoreType.DMA,
            pltpu.SemaphoreType.DMA,
        ],
    )
    def k(emb_hbm, kgg_hbm, rel_hbm, scg_ids_hbm, kgg_ids_hbm, rel_ids_hbm,
          out_scg, out_kgg, out_rel, idx_v, rows_v, rows_b, rel_sh,
          isem, rsem, gsem, ksem):
        sid = lax.axis_index("s")
        wid = sid * NC + lax.axis_index("c")
        base = wid * b_per_w

        @pl.when(sid == 0)
        def _():
            pltpu.async_copy(rel_hbm, rel_sh, rsem)

        id_copies = [
            pltpu.async_copy(ids_hbm.at[pl.ds(base, b_per_w)],
                             idx_v.at[pl.ds(t * b_per_w, b_per_w)], isem)
            for t, ids_hbm in enumerate(
                (scg_ids_hbm, kgg_ids_hbm, rel_ids_hbm))
        ]
        for cp in id_copies:
            cp.wait()

        def do_table(table_ref, out_hbm, t):
            pltpu.async_copy(
                table_ref.at[idx_v.at[pl.ds(t * b_per_w, b_per_w)]],
                rows_v, gsem).wait()
            pltpu.sync_copy(rows_v, out_hbm.at[pl.ds(base, b_per_w)])

        do_table(emb_hbm, out_scg, 0)

        pltpu.async_copy(
            kgg_hbm.at[idx_v.at[pl.ds(b_per_w, b_per_w)]],
            rows_v, gsem).wait()
        kgg_sc = pltpu.async_copy(
            rows_v, out_kgg.at[pl.ds(base, b_per_w)], ksem)

        @pl.when(sid == 0)
        def _():
            pltpu.make_async_copy(rel_hbm, rel_sh, rsem).wait()
        plsc.subcore_barrier()
        half = b_per_w // 2
        for c in range(2):
            pltpu.async_copy(
                rel_sh.at[idx_v.at[pl.ds(2 * b_per_w + c * half, half)]],
                rows_b, gsem).wait()
            pltpu.sync_copy(
                rows_b, out_rel.at[pl.ds(base + c * half, half)])
        kgg_sc.wait()

    return k


def kernel(embedding, kgg_table, rel_table, scg_ids, relation_ids, kgg_ids):
    B = scg_ids.shape[0]
    D = embedding.shape[1]
    R = rel_table.shape[0]
    info = plsc.get_sparse_core_info()
    NC, NS = info.num_cores, info.num_subcores
    k = _gather3(B, D, NC, NS, R)
    if scg_ids.dtype != jnp.int32:
        scg_ids = scg_ids.astype(jnp.int32)
        relation_ids = relation_ids.astype(jnp.int32)
        kgg_ids = kgg_ids.astype(jnp.int32)
    scg, kgg, rel = k(embedding, kgg_table, rel_table,
                      scg_ids, kgg_ids, relation_ids)
    return (scg, kgg, rel)


# speedup vs baseline: 1.0630x; 1.0203x over previous
import functools

import jax
import jax.numpy as jnp
from jax import lax
from jax.experimental import pallas as pl
from jax.experimental.pallas import tpu as pltpu
from jax.experimental.pallas import tpu_sc as plsc


def _gather3(B, D, NC, NS, R):
    NW = NC * NS
    b_per_w = B // NW
    mesh = plsc.VectorSubcoreMesh(core_axis_name="c", subcore_axis_name="s")

    @functools.partial(
        pl.kernel,
        mesh=mesh,
        out_type=(
            jax.ShapeDtypeStruct((B, D), jnp.float32),
            jax.ShapeDtypeStruct((B, D), jnp.float32),
            jax.ShapeDtypeStruct((B, D), jnp.float32),
        ),
        scratch_types=[
            pltpu.VMEM((3 * b_per_w,), jnp.int32),
            pltpu.VMEM((b_per_w, D), jnp.float32),
            pltpu.VMEM((b_per_w // 2, D), jnp.float32),
            pltpu.VMEM_SHARED((R, D), jnp.float32),
            pltpu.SemaphoreType.DMA,
            pltpu.SemaphoreType.DMA,
            pltpu.SemaphoreType.DMA,
            pltpu.SemaphoreType.DMA,
            pltpu.SemaphoreType.DMA,
            pltpu.SemaphoreType.DMA,
        ],
    )
    def k(emb_hbm, kgg_hbm, rel_hbm, scg_ids_hbm, kgg_ids_hbm, rel_ids_hbm,
          out_scg, out_kgg, out_rel, idx_v, rows_v, rows_b, rel_sh,
          isem, rsem, gsem, esem, ksem, lsem):
        sid = lax.axis_index("s")
        wid = sid * NC + lax.axis_index("c")
        base = wid * b_per_w

        @pl.when(sid == 0)
        def _():
            pltpu.async_copy(rel_hbm, rel_sh, rsem)

        id_copies = [
            pltpu.async_copy(ids_hbm.at[pl.ds(base, b_per_w)],
                             idx_v.at[pl.ds(t * b_per_w, b_per_w)], isem)
            for t, ids_hbm in enumerate(
                (scg_ids_hbm, kgg_ids_hbm, rel_ids_hbm))
        ]
        for cp in id_copies:
            cp.wait()
        half = b_per_w // 2

        pltpu.async_copy(
            emb_hbm.at[idx_v.at[pl.ds(0, b_per_w)]], rows_v, gsem).wait()
        emb_sc = pltpu.async_copy(
            rows_v, out_scg.at[pl.ds(base, b_per_w)], esem)

        @pl.when(sid == 0)
        def _():
            pltpu.make_async_copy(rel_hbm, rel_sh, rsem).wait()
        plsc.subcore_barrier()
        pltpu.async_copy(
            rel_sh.at[idx_v.at[pl.ds(2 * b_per_w, half)]],
            rows_b, gsem).wait()
        rel_sc1 = pltpu.async_copy(
            rows_b, out_rel.at[pl.ds(base, half)], lsem)

        emb_sc.wait()
        pltpu.async_copy(
            kgg_hbm.at[idx_v.at[pl.ds(b_per_w, b_per_w)]],
            rows_v, gsem).wait()
        kgg_sc = pltpu.async_copy(
            rows_v, out_kgg.at[pl.ds(base, b_per_w)], ksem)

        rel_sc1.wait()
        pltpu.async_copy(
            rel_sh.at[idx_v.at[pl.ds(2 * b_per_w + half, half)]],
            rows_b, gsem).wait()
        pltpu.sync_copy(rows_b, out_rel.at[pl.ds(base + half, half)])
        kgg_sc.wait()

    return k


def kernel(embedding, kgg_table, rel_table, scg_ids, relation_ids, kgg_ids):
    B = scg_ids.shape[0]
    D = embedding.shape[1]
    R = rel_table.shape[0]
    info = plsc.get_sparse_core_info()
    NC, NS = info.num_cores, info.num_subcores
    k = _gather3(B, D, NC, NS, R)
    if scg_ids.dtype != jnp.int32:
        scg_ids = scg_ids.astype(jnp.int32)
        relation_ids = relation_ids.astype(jnp.int32)
        kgg_ids = kgg_ids.astype(jnp.int32)
    scg, kgg, rel = k(embedding, kgg_table, rel_table,
                      scg_ids, kgg_ids, relation_ids)
    return (scg, kgg, rel)
